# Initial kernel scaffold; baseline (speedup 1.0000x reference)
import jax
import jax.numpy as jnp
from jax.experimental import pallas as pl


def kernel(inputs, embedding_weight):
    raise NotImplementedError("write your pallas kernel here")


# speedup vs baseline: 3.6982x; 3.6982x over previous
import functools

import jax
import jax.numpy as jnp
from jax.experimental import pallas as pl

_K = 1024
_D = 64
_COMMIT = 0.25


def _vq_body(x_ref, e_ref, et_ref, q_ref, i_ref, loss_ref):
    b = pl.program_id(0)
    tb = pl.program_id(1)
    x = x_ref[0]
    emb = e_ref[...]
    embt = et_ref[...]

    xt = jnp.transpose(x)
    x_sq = jnp.sum(xt * xt, axis=1, keepdims=True)
    e_sq = jnp.sum(emb * emb, axis=1)
    xe = jax.lax.dot_general(
        xt, embt, (((1,), (0,)), ((), ())),
        preferred_element_type=jnp.float32)
    dist = x_sq + e_sq[None, :] - 2.0 * xe

    minv = jnp.min(dist, axis=1, keepdims=True)
    kiota = jax.lax.broadcasted_iota(jnp.int32, dist.shape, 1)
    idx = jnp.min(jnp.where(dist == minv, kiota, _K), axis=1)

    onehot_t = (jax.lax.broadcasted_iota(jnp.int32, (_K, x.shape[1]), 0)
                == idx[None, :]).astype(jnp.float32)
    quant = jax.lax.dot_general(
        embt, onehot_t, (((1,), (0,)), ((), ())),
        preferred_element_type=jnp.float32)

    q_ref[0] = x + (quant - x)
    i_ref[0, 0, 0] = idx

    part = jnp.reshape(jnp.sum((quant - x) ** 2), (1, 1))

    @pl.when((b == 0) & (tb == 0))
    def _init():
        loss_ref[...] = jnp.zeros((1, 1), jnp.float32)

    loss_ref[...] += part


@functools.partial(jax.jit, static_argnames=("tblk",))
def _vq(inputs, embedding_weight, tblk=512):
    B, C, T = inputs.shape
    nt = T // tblk
    embt = jnp.transpose(embedding_weight, (1, 0))

    quant, idx4, loss = pl.pallas_call(
        _vq_body,
        grid=(B, nt),
        in_specs=[
            pl.BlockSpec((1, C, tblk), lambda b, t: (b, 0, t)),
            pl.BlockSpec((_K, _D), lambda b, t: (0, 0)),
            pl.BlockSpec((_D, _K), lambda b, t: (0, 0)),
        ],
        out_specs=[
            pl.BlockSpec((1, C, tblk), lambda b, t: (b, 0, t)),
            pl.BlockSpec((1, 1, 1, tblk), lambda b, t: (b, t, 0, 0)),
            pl.BlockSpec((1, 1), lambda b, t: (0, 0)),
        ],
        out_shape=[
            jax.ShapeDtypeStruct((B, C, T), jnp.float32),
            jax.ShapeDtypeStruct((B, nt, 1, tblk), jnp.int32),
            jax.ShapeDtypeStruct((1, 1), jnp.float32),
        ],
    )(inputs, embedding_weight, embt)

    indices = idx4.reshape(B, T)
    m = loss[0, 0] / (B * T * C)
    loss = m + _COMMIT * m
    return quant, loss, indices


def kernel(inputs, embedding_weight):
    return _vq(inputs, embedding_weight)
